# Initial kernel scaffold; baseline (speedup 1.0000x reference)
import jax
import jax.numpy as jnp
from jax.experimental import pallas as pl


def kernel(x, edge_index, edge_attr, Wl1, bl1, Wr1, br1, We1, att1, b1, Wl2, bl2, Wr2, br2, We2, att2, b2, Wl3, bl3, Wr3, br3, We3, att3, b3, Wout, bout):
    raise NotImplementedError("write your pallas kernel here")


# speedup vs baseline: 6.7529x; 6.7529x over previous
import functools

import jax
import jax.numpy as jnp
from jax import lax
from jax.experimental import pallas as pl
from jax.experimental.pallas import tpu as pltpu
from jax.experimental.pallas import tpu_sc as plsc

N_NODES = 10000
D = 128
NT = 32
NPT = 320
NPAD = NT * NPT
CHUNK = 128
NEG_BIG = -1e30


def _mm_body(h_ref, wl_ref, bl_ref, wr_ref, br_ref, xl_ref, xr_ref):
    h = h_ref[...]
    xl_ref[...] = jnp.dot(h, wl_ref[...], preferred_element_type=jnp.float32) + bl_ref[...]
    xr_ref[...] = jnp.dot(h, wr_ref[...], preferred_element_type=jnp.float32) + br_ref[...]


def _matmuls(h, Wl, bl, Wr, br):
    RB = 2560
    grid = NPAD // RB
    return pl.pallas_call(
        _mm_body,
        grid=(grid,),
        in_specs=[
            pl.BlockSpec((RB, D), lambda i: (i, 0)),
            pl.BlockSpec((D, D), lambda i: (0, 0)),
            pl.BlockSpec((1, D), lambda i: (0, 0)),
            pl.BlockSpec((D, D), lambda i: (0, 0)),
            pl.BlockSpec((1, D), lambda i: (0, 0)),
        ],
        out_specs=[pl.BlockSpec((RB, D), lambda i: (i, 0)),
                   pl.BlockSpec((RB, D), lambda i: (i, 0))],
        out_shape=[jax.ShapeDtypeStruct((NPAD, D), jnp.float32),
                   jax.ShapeDtypeStruct((NPAD, D), jnp.float32)],
    )(h, Wl, bl.reshape(1, D), Wr, br.reshape(1, D))


def _sc_layer_body(last, xl_hbm, xr_hbm, srcs_hbm, dsts_hbm, eas_hbm,
                   est_hbm, wv_hbm, out_hbm,
                   xr_loc, acc, amaxv, denv, glbuf, sidx, dbuf, eabuf,
                   est_v, wv, ybuf, sem):
    cid = lax.axis_index("c")
    sid = lax.axis_index("s")
    t = sid * 2 + cid
    n0 = t * NPT

    pltpu.sync_copy(est_hbm, est_v)
    pltpu.sync_copy(wv_hbm, wv)
    pltpu.sync_copy(xr_hbm.at[pl.ds(n0, NPT)], xr_loc)

    zero16 = jnp.zeros((16,), jnp.float32)
    ninf16 = jnp.full((16,), NEG_BIG, jnp.float32)

    def init_body(n, _):
        amaxv[n, :] = ninf16
        denv[n, :] = zero16
        for j in range(8):
            acc[n, pl.ds(j * 16, 16)] = zero16
        return 0

    lax.fori_loop(0, NPT, init_body, 0)

    ev = est_v[pl.ds(t, 16)]
    e0 = ev[0]
    e1 = ev[1]
    ecb0 = e0 - lax.rem(e0, CHUNK)
    nch = (e1 - ecb0 + CHUNK - 1) // CHUNK

    def chunk_body(ci, _):
        ecb = pl.multiple_of(ecb0 + ci * CHUNK, CHUNK)
        pltpu.sync_copy(srcs_hbm.at[pl.ds(ecb, CHUNK)], sidx)
        pltpu.sync_copy(dsts_hbm.at[pl.ds(ecb, CHUNK)], dbuf.at[pl.ds(0, CHUNK)])
        pltpu.sync_copy(eas_hbm.at[pl.ds(ecb, CHUNK)], eabuf.at[pl.ds(0, CHUNK)])
        pltpu.async_copy(xl_hbm.at[sidx], glbuf, sem).wait()
        lo = jnp.maximum(e0, ecb) - ecb
        hi = jnp.minimum(e1, ecb + CHUNK) - ecb

        def edge_body(el, _):
            dv = dbuf[pl.ds(el, 16)]
            eav = eabuf[pl.ds(el, 16)]
            cv = jnp.where(eav < 0.0, 0.0,
                           jnp.where(eav == 0.0, 1e4, 1.0 / eav))
            dl = dv[0] - n0
            c = cv[0]
            sacc = jnp.zeros((16,), jnp.float32)
            glr = []
            for j in range(8):
                glj = glbuf[el, pl.ds(j * 16, 16)]
                xrj = xr_loc[dl, pl.ds(j * 16, 16)]
                wej = wv[0, pl.ds(j * 16, 16)]
                attj = wv[1, pl.ds(j * 16, 16)]
                u = glj + xrj + c * wej
                lr = jnp.maximum(u, 0.2 * u)
                sacc = sacc + attj * lr
                glr.append(glj)
            alpha = jnp.sum(sacc)
            av = jnp.full((16,), alpha)
            mo = amaxv[dl, :]
            mn = jnp.maximum(mo, av)
            scv = jnp.exp(mo - mn)
            aev = jnp.exp(av - mn)
            amaxv[dl, :] = mn
            denv[dl, :] = denv[dl, :] * scv + aev
            for j in range(8):
                accj = acc[dl, pl.ds(j * 16, 16)]
                acc[dl, pl.ds(j * 16, 16)] = accj * scv + aev * glr[j]
            return 0

        lax.fori_loop(lo, hi, edge_body, 0)
        return 0

    lax.fori_loop(0, nch, chunk_body, 0)

    def node_body(n, _):
        r = 1.0 / (denv[n, :] + 1e-16)
        if not last:
            for j in range(8):
                bj = wv[2, pl.ds(j * 16, 16)]
                row = acc[n, pl.ds(j * 16, 16)] * r + bj
                acc[n, pl.ds(j * 16, 16)] = jnp.maximum(row, 0.0)
        else:
            yv = jnp.zeros((16,), jnp.float32)
            for j in range(8):
                bj = wv[2, pl.ds(j * 16, 16)]
                row = acc[n, pl.ds(j * 16, 16)] * r + bj
                yv = yv + row * wv[3, pl.ds(j * 16, 16)]
            boutv = wv[4, pl.ds(0, 16)]
            ybuf[n, :] = jnp.full((16,), jnp.sum(yv)) + boutv
        return 0

    lax.fori_loop(0, NPT, node_body, 0)

    if not last:
        pltpu.sync_copy(acc, out_hbm.at[pl.ds(n0, NPT)])
    else:
        pltpu.sync_copy(ybuf, out_hbm.at[t])


def _sc_layer(last, xl, xr, srcs, dsts, eas, estart, wvec):
    mesh = plsc.VectorSubcoreMesh(core_axis_name="c", subcore_axis_name="s")
    if last:
        out_type = jax.ShapeDtypeStruct((NT, NPT, 16), jnp.float32)
    else:
        out_type = jax.ShapeDtypeStruct((NPAD, D), jnp.float32)
    fn = pl.kernel(
        functools.partial(_sc_layer_body, last),
        out_type=out_type,
        mesh=mesh,
        scratch_types=[
            pltpu.VMEM((NPT, D), jnp.float32),
            pltpu.VMEM((NPT, D), jnp.float32),
            pltpu.VMEM((NPT, 16), jnp.float32),
            pltpu.VMEM((NPT, 16), jnp.float32),
            pltpu.VMEM((CHUNK, D), jnp.float32),
            pltpu.VMEM((CHUNK,), jnp.int32),
            pltpu.VMEM((CHUNK + 16,), jnp.int32),
            pltpu.VMEM((CHUNK + 16,), jnp.float32),
            pltpu.VMEM((48,), jnp.int32),
            pltpu.VMEM((5, D), jnp.float32),
            pltpu.VMEM((NPT, 16), jnp.float32),
            pltpu.SemaphoreType.DMA,
        ],
        compiler_params=pltpu.CompilerParams(
            needs_layout_passes=False, use_tc_tiling_on_sc=False),
    )
    return fn(xl, xr, srcs, dsts, eas, estart, wvec)


def kernel(x, edge_index, edge_attr, Wl1, bl1, Wr1, br1, We1, att1, b1,
           Wl2, bl2, Wr2, br2, We2, att2, b2, Wl3, bl3, Wr3, br3, We3,
           att3, b3, Wout, bout):
    src = edge_index[0]
    dst = edge_index[1]
    e = src.shape[0]
    sl = jnp.arange(N_NODES, dtype=src.dtype)
    src2 = jnp.concatenate([src, sl])
    dst2 = jnp.concatenate([dst, sl])
    ea2 = jnp.concatenate([edge_attr[:, 0],
                           jnp.full((N_NODES,), -1.0, jnp.float32)])
    e2 = e + N_NODES
    bins = dst2 // NPT
    order = jnp.argsort(bins)
    bs = bins[order]
    estart = jnp.searchsorted(bs, jnp.arange(NT + 1, dtype=jnp.int32)
                              ).astype(jnp.int32)
    estart = jnp.concatenate(
        [estart, jnp.full((48 - NT - 1,), e2, jnp.int32)])
    e3 = ((e2 + CHUNK - 1) // CHUNK + 1) * CHUNK
    padn = e3 - e2
    srcs = jnp.concatenate([src2[order], jnp.zeros((padn,), src.dtype)])
    dsts = jnp.concatenate([dst2[order], jnp.zeros((padn,), dst.dtype)])
    eas = jnp.concatenate([ea2[order], jnp.full((padn,), -1.0, jnp.float32)])

    h = jnp.zeros((NPAD, D), jnp.float32).at[:N_NODES].set(x)
    for (Wl, bl, Wr, br, We, att, b, last) in (
            (Wl1, bl1, Wr1, br1, We1, att1, b1, False),
            (Wl2, bl2, Wr2, br2, We2, att2, b2, False),
            (Wl3, bl3, Wr3, br3, We3, att3, b3, True)):
        xl, xr = _matmuls(h, Wl, bl, Wr, br)
        wvec = jnp.stack([
            We[0], att, b,
            Wout[:, 0] if last else jnp.zeros((D,), jnp.float32),
            jnp.full((D,), bout[0] if last else 0.0, jnp.float32)])
        h = _sc_layer(last, xl, xr, srcs, dsts, eas, estart, wvec)
    return h.reshape(NPAD, 16)[:N_NODES, 0]
